# Initial kernel scaffold; baseline (speedup 1.0000x reference)
import jax
import jax.numpy as jnp
from jax.experimental import pallas as pl


def kernel(x, edge_index, edge_weights, Ws0, Wn0, b0, Ws1, Wn1, b1, Ws2, Wn2, b2):
    raise NotImplementedError("write your pallas kernel here")


# speedup vs baseline: 2.5396x; 2.5396x over previous
import functools

import jax
import jax.numpy as jnp
from jax import lax
from jax.experimental import pallas as pl
from jax.experimental.pallas import tpu as pltpu
from jax.experimental.pallas import tpu_sc as plsc

N = 10000
NP = 10240
E = 320000
NC, NS = 2, 16
NW = NC * NS
L = 16
CE = 4000
BN = 2048
NBLK = NP // BN


@functools.lru_cache(maxsize=None)
def _make_agg(D: int, with_wsum: bool):
    dpw = D // NW
    nchunks = E // CE
    ngroups = CE // L
    mesh = plsc.VectorSubcoreMesh(
        core_axis_name="c", subcore_axis_name="s", num_cores=NC, num_subcores=NS
    )
    out_type = [jax.ShapeDtypeStruct((D * NP,), jnp.float32)]
    scratch = [
        pltpu.VMEM((dpw * NP,), jnp.float32),
        pltpu.VMEM((dpw * NP,), jnp.float32),
        pltpu.VMEM((CE,), jnp.int32),
        pltpu.VMEM((CE,), jnp.int32),
        pltpu.VMEM((CE,), jnp.float32),
    ]
    if with_wsum:
        out_type.append(jax.ShapeDtypeStruct((NP,), jnp.float32))
        scratch.append(pltpu.VMEM((NP,), jnp.float32))

    @functools.partial(
        pl.kernel, out_type=tuple(out_type), mesh=mesh, scratch_types=scratch,
        compiler_params=pltpu.CompilerParams(needs_layout_passes=False),
    )
    def agg(*refs):
        if with_wsum:
            (p_hbm, src_hbm, dst_hbm, w_hbm, out_hbm, wsum_hbm,
             pcols, ocols, src_v, dst_v, w_v, wacc) = refs
        else:
            (p_hbm, src_hbm, dst_hbm, w_hbm, out_hbm,
             pcols, ocols, src_v, dst_v, w_v) = refs
            wacc = None
        wid = lax.axis_index("s") * NC + lax.axis_index("c")
        base = wid * dpw * NP
        pltpu.sync_copy(p_hbm.at[pl.ds(base, dpw * NP)], pcols)

        zero = jnp.zeros((L,), jnp.float32)

        def zero_body(i, _):
            ocols[pl.ds(i * L, L)] = zero
            if with_wsum:
                @pl.when(i < NP // L)
                def _():
                    wacc[pl.ds(i * L, L)] = zero
            return 0

        lax.fori_loop(0, dpw * NP // L, zero_body, 0)

        def chunk_body(k, _):
            e0 = k * CE
            pltpu.sync_copy(src_hbm.at[pl.ds(e0, CE)], src_v)
            pltpu.sync_copy(dst_hbm.at[pl.ds(e0, CE)], dst_v)
            pltpu.sync_copy(w_hbm.at[pl.ds(e0, CE)], w_v)

            def group_body(g, _):
                off = g * L
                s = src_v[pl.ds(off, L)]
                d = dst_v[pl.ds(off, L)]
                wv = w_v[pl.ds(off, L)]
                for c in range(dpw):
                    vals = plsc.load_gather(pcols, [s + (c * NP)])
                    plsc.addupdate_scatter(ocols, [d + (c * NP)], vals * wv)
                if with_wsum:
                    plsc.addupdate_scatter(wacc, [d], wv)
                return 0

            lax.fori_loop(0, ngroups, group_body, 0)
            return 0

        lax.fori_loop(0, nchunks, chunk_body, 0)

        pltpu.sync_copy(ocols, out_hbm.at[pl.ds(base, dpw * NP)])
        if with_wsum:
            @pl.when(wid == 0)
            def _():
                pltpu.sync_copy(wacc, wsum_hbm)

    return agg


def _agg_wsum(p, src, dst, w):
    D = p.shape[0]
    out, wsum = _make_agg(D, True)(p.reshape(-1), src, dst, w)
    return out.reshape(D, NP), wsum


def _agg(p, src, dst, w):
    D = p.shape[0]
    (out,) = _make_agg(D, False)(p.reshape(-1), src, dst, w)
    return out.reshape(D, NP)


def _proj(W, h):
    return lax.dot_general(
        W, h, (((0,), (0,)), ((), ())), preferred_element_type=jnp.float32
    )


def _combine(s, agg, winv, res, relu):
    t = s + agg * winv
    if relu:
        t = jnp.maximum(t, 0.0)
    if res is not None:
        t = t + res
    nrm = jnp.sqrt(jnp.sum(t * t, axis=0, keepdims=True))
    return t / jnp.maximum(nrm, 1e-12)


def _winv(wsum_blk):
    return 1.0 / jnp.maximum(wsum_blk[0], 1e-6)


def _bspec(d):
    return pl.BlockSpec((d, BN), lambda i: (0, i))


_WSPEC = pl.BlockSpec((1, 1, BN), lambda i: (i, 0, 0))


def _full(shape):
    return pl.BlockSpec(shape, lambda i: tuple(0 for _ in shape))


def _tc_first(xT, Ws, Wn, b2):
    din, dout = Ws.shape

    def body(x_ref, ws_ref, wn_ref, b_ref, s_ref, p_ref):
        h = x_ref[...]
        s_ref[...] = _proj(ws_ref[...], h) + b_ref[:, :1]
        p_ref[...] = _proj(wn_ref[...], h)

    return pl.pallas_call(
        body,
        grid=(NBLK,),
        in_specs=[_bspec(din), _full((din, dout)), _full((din, dout)),
                  _full((dout, 128))],
        out_specs=[_bspec(dout), _bspec(dout)],
        out_shape=[jax.ShapeDtypeStruct((dout, NP), jnp.float32)] * 2,
    )(xT, Ws, Wn, b2)


def _tc_mid(s, agg, wsum3, res, Ws, Wn, b2, relu, emit_h):
    din, dout = Ws.shape
    have_res = res is not None

    def body(*refs):
        if have_res:
            s_ref, a_ref, w_ref, r_ref, ws_ref, wn_ref, b_ref, *outs = refs
            rv = r_ref[...]
        else:
            s_ref, a_ref, w_ref, ws_ref, wn_ref, b_ref, *outs = refs
            rv = None
        h = _combine(s_ref[...], a_ref[...], _winv(w_ref[...]), rv, relu)
        outs[0][...] = _proj(ws_ref[...], h) + b_ref[:, :1]
        outs[1][...] = _proj(wn_ref[...], h)
        if emit_h:
            outs[2][...] = h

    in_specs = [_bspec(din), _bspec(din), _WSPEC]
    args = [s, agg, wsum3]
    if have_res:
        in_specs.append(_bspec(din))
        args.append(res)
    in_specs += [_full((din, dout)), _full((din, dout)), _full((dout, 128))]
    args += [Ws, Wn, b2]
    n_out = 3 if emit_h else 2
    out_specs = [_bspec(dout), _bspec(dout)] + ([_bspec(din)] if emit_h else [])
    out_shape = ([jax.ShapeDtypeStruct((dout, NP), jnp.float32)] * 2
                 + ([jax.ShapeDtypeStruct((din, NP), jnp.float32)] if emit_h else []))
    return pl.pallas_call(
        body, grid=(NBLK,), in_specs=in_specs, out_specs=out_specs,
        out_shape=out_shape,
    )(*args)


def _tc_last(s, agg, wsum3, res):
    d = s.shape[0]

    def body(s_ref, a_ref, w_ref, r_ref, o_ref):
        o_ref[...] = _combine(
            s_ref[...], a_ref[...], _winv(w_ref[...]), r_ref[...], relu=False
        )

    return pl.pallas_call(
        body,
        grid=(NBLK,),
        in_specs=[_bspec(d), _bspec(d), _WSPEC, _bspec(d)],
        out_specs=_bspec(d),
        out_shape=jax.ShapeDtypeStruct((d, NP), jnp.float32),
    )(s, agg, wsum3, res)


def kernel(x, edge_index, edge_weights, Ws0, Wn0, b0, Ws1, Wn1, b1, Ws2, Wn2, b2):
    xT = jnp.pad(x.T, ((0, 0), (0, NP - N)))
    src = edge_index[0].astype(jnp.int32)
    dst = edge_index[1].astype(jnp.int32)
    w = edge_weights.astype(jnp.float32)
    b0r = jnp.tile(b0[:, None], (1, 128))
    b1r = jnp.tile(b1[:, None], (1, 128))
    b2r = jnp.tile(b2[:, None], (1, 128))

    s0, p0 = _tc_first(xT, Ws0, Wn0, b0r)
    agg0, wsum = _agg_wsum(p0, src, dst, w)
    wsum3 = wsum.reshape(NBLK, 1, BN)
    s1, p1 = _tc_mid(s0, agg0, wsum3, xT, Ws1, Wn1, b1r, relu=True, emit_h=False)
    agg1 = _agg(p1, src, dst, w)
    s2, p2, h2 = _tc_mid(s1, agg1, wsum3, None, Ws2, Wn2, b2r, relu=True,
                         emit_h=True)
    agg2 = _agg(p2, src, dst, w)
    h3 = _tc_last(s2, agg2, wsum3, h2)
    return h3[:, :N].T


# speedup vs baseline: 6.6177x; 2.6058x over previous
import functools

import jax
import jax.numpy as jnp
from jax import lax
from jax.experimental import pallas as pl
from jax.experimental.pallas import tpu as pltpu
from jax.experimental.pallas import tpu_sc as plsc

N = 10000
NP = 10240
E = 320000
NC, NS = 2, 16
NW = NC * NS
L = 16
CE = 8000
BN = 2048
NBLK = NP // BN


@functools.lru_cache(maxsize=None)
def _make_agg(D: int, with_wsum: bool):
    dpw = D // NW
    nchunks = E // CE
    ngroups = CE // L
    mesh = plsc.VectorSubcoreMesh(
        core_axis_name="c", subcore_axis_name="s", num_cores=NC, num_subcores=NS
    )
    out_type = [jax.ShapeDtypeStruct((D * NP,), jnp.float32)]
    scratch = [
        pltpu.VMEM((dpw * NP,), jnp.float32),
        pltpu.VMEM((dpw * NP,), jnp.float32),
        pltpu.VMEM((CE,), jnp.int32),
        pltpu.VMEM((CE,), jnp.float32),
    ]
    if with_wsum:
        out_type.append(jax.ShapeDtypeStruct((NP,), jnp.float32))
        scratch.append(pltpu.VMEM((NP,), jnp.float32))

    @functools.partial(
        pl.kernel, out_type=tuple(out_type), mesh=mesh, scratch_types=scratch,
        compiler_params=pltpu.CompilerParams(needs_layout_passes=False),
    )
    def agg(*refs):
        if with_wsum:
            (p_hbm, sd_hbm, w_hbm, out_hbm, wsum_hbm,
             pcols, ocols, sd_v, w_v, wacc) = refs
        else:
            (p_hbm, sd_hbm, w_hbm, out_hbm,
             pcols, ocols, sd_v, w_v) = refs
            wacc = None
        wid = lax.axis_index("s") * NC + lax.axis_index("c")
        base = wid * dpw * NP
        pltpu.sync_copy(p_hbm.at[pl.ds(base, dpw * NP)], pcols)

        zero = jnp.zeros((L,), jnp.float32)

        @plsc.parallel_loop(0, dpw * NP // L, unroll=8)
        def _(i):
            ocols[pl.ds(i * L, L)] = zero

        if with_wsum:
            @plsc.parallel_loop(0, NP // L, unroll=8)
            def _(i):
                wacc[pl.ds(i * L, L)] = zero

        def chunk_body(k, _):
            e0 = k * CE
            pltpu.sync_copy(sd_hbm.at[pl.ds(e0, CE)], sd_v)
            pltpu.sync_copy(w_hbm.at[pl.ds(e0, CE)], w_v)

            @plsc.parallel_loop(0, ngroups, unroll=4)
            def _(g):
                off = g * L
                sd = sd_v[pl.ds(off, L)]
                s = lax.bitwise_and(sd, jnp.int32(0x3FFF))
                d = lax.shift_right_logical(sd, jnp.int32(14))
                wv = w_v[pl.ds(off, L)]
                for c in range(dpw):
                    vals = plsc.load_gather(pcols, [s + (c * NP)])
                    plsc.addupdate_scatter(ocols, [d + (c * NP)], vals * wv)
                if with_wsum:
                    plsc.addupdate_scatter(wacc, [d], wv)

            return 0

        lax.fori_loop(0, nchunks, chunk_body, 0)

        pltpu.sync_copy(ocols, out_hbm.at[pl.ds(base, dpw * NP)])
        if with_wsum:
            @pl.when(wid == 0)
            def _():
                pltpu.sync_copy(wacc, wsum_hbm)

    return agg


def _agg_wsum(p, sd, w):
    D = p.shape[0]
    out, wsum = _make_agg(D, True)(p.reshape(-1), sd, w)
    return out.reshape(D, NP), wsum


def _agg(p, sd, w):
    D = p.shape[0]
    (out,) = _make_agg(D, False)(p.reshape(-1), sd, w)
    return out.reshape(D, NP)


def _proj(W, h):
    return lax.dot_general(
        W, h, (((0,), (0,)), ((), ())), preferred_element_type=jnp.float32
    )


def _combine(s, agg, winv, res, relu):
    t = s + agg * winv
    if relu:
        t = jnp.maximum(t, 0.0)
    if res is not None:
        t = t + res
    nrm = jnp.sqrt(jnp.sum(t * t, axis=0, keepdims=True))
    return t / jnp.maximum(nrm, 1e-12)


def _winv(wsum_blk):
    return 1.0 / jnp.maximum(wsum_blk[0], 1e-6)


def _bspec(d):
    return pl.BlockSpec((d, BN), lambda i: (0, i))


_WSPEC = pl.BlockSpec((1, 1, BN), lambda i: (i, 0, 0))


def _full(shape):
    return pl.BlockSpec(shape, lambda i: tuple(0 for _ in shape))


def _tc_first(xT, Ws, Wn, b2):
    din, dout = Ws.shape

    def body(x_ref, ws_ref, wn_ref, b_ref, s_ref, p_ref):
        h = x_ref[...]
        s_ref[...] = _proj(ws_ref[...], h) + b_ref[:, :1]
        p_ref[...] = _proj(wn_ref[...], h)

    return pl.pallas_call(
        body,
        grid=(NBLK,),
        in_specs=[_bspec(din), _full((din, dout)), _full((din, dout)),
                  _full((dout, 128))],
        out_specs=[_bspec(dout), _bspec(dout)],
        out_shape=[jax.ShapeDtypeStruct((dout, NP), jnp.float32)] * 2,
    )(xT, Ws, Wn, b2)


def _tc_mid(s, agg, wsum3, res, Ws, Wn, b2, relu, emit_h):
    din, dout = Ws.shape
    have_res = res is not None

    def body(*refs):
        if have_res:
            s_ref, a_ref, w_ref, r_ref, ws_ref, wn_ref, b_ref, *outs = refs
            rv = r_ref[...]
        else:
            s_ref, a_ref, w_ref, ws_ref, wn_ref, b_ref, *outs = refs
            rv = None
        h = _combine(s_ref[...], a_ref[...], _winv(w_ref[...]), rv, relu)
        outs[0][...] = _proj(ws_ref[...], h) + b_ref[:, :1]
        outs[1][...] = _proj(wn_ref[...], h)
        if emit_h:
            outs[2][...] = h

    in_specs = [_bspec(din), _bspec(din), _WSPEC]
    args = [s, agg, wsum3]
    if have_res:
        in_specs.append(_bspec(din))
        args.append(res)
    in_specs += [_full((din, dout)), _full((din, dout)), _full((dout, 128))]
    args += [Ws, Wn, b2]
    n_out = 3 if emit_h else 2
    out_specs = [_bspec(dout), _bspec(dout)] + ([_bspec(din)] if emit_h else [])
    out_shape = ([jax.ShapeDtypeStruct((dout, NP), jnp.float32)] * 2
                 + ([jax.ShapeDtypeStruct((din, NP), jnp.float32)] if emit_h else []))
    return pl.pallas_call(
        body, grid=(NBLK,), in_specs=in_specs, out_specs=out_specs,
        out_shape=out_shape,
    )(*args)


def _tc_last(s, agg, wsum3, res):
    d = s.shape[0]

    def body(s_ref, a_ref, w_ref, r_ref, o_ref):
        o_ref[...] = _combine(
            s_ref[...], a_ref[...], _winv(w_ref[...]), r_ref[...], relu=False
        )

    return pl.pallas_call(
        body,
        grid=(NBLK,),
        in_specs=[_bspec(d), _bspec(d), _WSPEC, _bspec(d)],
        out_specs=_bspec(d),
        out_shape=jax.ShapeDtypeStruct((d, NP), jnp.float32),
    )(s, agg, wsum3, res)


def kernel(x, edge_index, edge_weights, Ws0, Wn0, b0, Ws1, Wn1, b1, Ws2, Wn2, b2):
    xT = jnp.pad(x.T, ((0, 0), (0, NP - N)))
    src = edge_index[0].astype(jnp.int32)
    dst = edge_index[1].astype(jnp.int32)
    sd = src | (dst << 14)
    w = edge_weights.astype(jnp.float32)
    b0r = jnp.tile(b0[:, None], (1, 128))
    b1r = jnp.tile(b1[:, None], (1, 128))
    b2r = jnp.tile(b2[:, None], (1, 128))

    s0, p0 = _tc_first(xT, Ws0, Wn0, b0r)
    agg0, wsum = _agg_wsum(p0, sd, w)
    wsum3 = wsum.reshape(NBLK, 1, BN)
    s1, p1 = _tc_mid(s0, agg0, wsum3, xT, Ws1, Wn1, b1r, relu=True, emit_h=False)
    agg1 = _agg(p1, sd, w)
    s2, p2, h2 = _tc_mid(s1, agg1, wsum3, None, Ws2, Wn2, b2r, relu=True,
                         emit_h=True)
    agg2 = _agg(p2, sd, w)
    h3 = _tc_last(s2, agg2, wsum3, h2)
    return h3[:, :N].T
